# Initial kernel scaffold; baseline (speedup 1.0000x reference)
import jax
import jax.numpy as jnp
from jax.experimental import pallas as pl


def kernel(x, beta, node_parent, edge_src, edge_dst):
    raise NotImplementedError("write your pallas kernel here")


# speedup vs baseline: 29.6123x; 29.6123x over previous
import functools

import jax
import jax.numpy as jnp
from jax import lax
from jax.experimental import pallas as pl
from jax.experimental.pallas import tpu as pltpu
from jax.experimental.pallas import tpu_sc as plsc

N = 100000
P = 6250
E = 1600000
QMIN = 0.1
S_B = 1.0

NC = 2
NS = 16
NW = NC * NS
L = 16

NODE_PER_TILE = 3136
N_PAD = NODE_PER_TILE * NW
ROWS0 = N_PAD // 128

P_PER_TILE = 224
P_PAD = P_PER_TILE * NW
PCHUNKS = P_PER_TILE // L

EDGES_PER_TILE = E // NW
ECHUNK = 80
NECHUNK = EDGES_PER_TILE // ECHUNK
ESUB = ECHUNK // L

BIG_I = 1 << 30

_mesh = plsc.VectorSubcoreMesh(core_axis_name="c", subcore_axis_name="s")


def _wid():
    return lax.axis_index("c") * NS + lax.axis_index("s")


def _lex_better(b_new, i_new, b_cur, i_cur):
    return (b_new > b_cur) | ((b_new == b_cur) & (i_new < i_cur))


def _prep_body(beta_ref, q_ref, bsum_ref):
    b = beta_ref[...]
    rows = lax.broadcasted_iota(jnp.int32, b.shape, 0)
    cols = lax.broadcasted_iota(jnp.int32, b.shape, 1)
    valid = (rows * 128 + cols) < N
    bc = jnp.clip(b, 0.0, 1.0 - 1e-4)
    at = 0.5 * jnp.log((1.0 + bc) / (1.0 - bc))
    q = at * at + QMIN
    q_ref[...] = jnp.where(valid, q, 0.0)
    bsum_ref[...] = jnp.sum(jnp.where(valid, bc, 0.0)).reshape(1, 1)


def _prep(beta2d):
    return pl.pallas_call(
        _prep_body,
        out_shape=[
            jax.ShapeDtypeStruct((ROWS0, 128), jnp.float32),
            jax.ShapeDtypeStruct((1, 1), jnp.float32),
        ],
    )(beta2d)


def _node_pass_body(beta_hbm, par_hbm, outb_hbm, outi_hbm,
                    beta_s, par_s, tab_b, tab_i, pbuf_f, pbuf_i, ibuf):
    wid = _wid()
    base = wid * NODE_PER_TILE
    pltpu.sync_copy(beta_hbm.at[pl.ds(base, NODE_PER_TILE)], beta_s)
    pltpu.sync_copy(par_hbm.at[pl.ds(base, NODE_PER_TILE)], par_s)

    iota = lax.iota(jnp.int32, L)
    neg1 = jnp.full((L,), -1.0, jnp.float32)
    big = jnp.full((L,), BIG_I, jnp.int32)

    def init_body(k, carry):
        tab_b[pl.ds(k * L, L)] = neg1
        tab_i[pl.ds(k * L, L)] = big
        return carry

    lax.fori_loop(0, P_PAD // L, init_body, 0)

    def body(ci, carry):
        off = ci * L
        b_raw = beta_s[pl.ds(off, L)]
        p_raw = par_s[pl.ds(off, L)]
        gidx = base + off + iota
        valid = gidx < N
        b16 = jnp.where(valid, jnp.clip(b_raw, 0.0, 1.0 - 1e-4), -1.0)
        p16 = jnp.where(valid, p_raw, 0)
        pbuf_f[...] = b16
        pbuf_i[...] = p16
        ibuf[...] = gidx
        lose = jnp.zeros((L,), jnp.bool_)
        for j in range(L):
            sj = jnp.full((L,), j, jnp.int32)
            pj = plsc.load_gather(pbuf_i, [sj])
            bj = plsc.load_gather(pbuf_f, [sj])
            ij = plsc.load_gather(ibuf, [sj])
            lose = lose | ((pj == p16) & _lex_better(bj, ij, b16, gidx))
        tb = plsc.load_gather(tab_b, [p16])
        ti = plsc.load_gather(tab_i, [p16])
        upd = (~lose) & _lex_better(b16, gidx, tb, ti)
        plsc.store_scatter(tab_b, [p16], b16, mask=upd)
        plsc.store_scatter(tab_i, [p16], gidx, mask=upd)
        return carry

    lax.fori_loop(0, NODE_PER_TILE // L, body, 0)
    pltpu.sync_copy(tab_b, outb_hbm.at[wid])
    pltpu.sync_copy(tab_i, outi_hbm.at[wid])


def _node_pass(beta_pad, par_pad):
    return pl.kernel(
        _node_pass_body,
        out_type=[
            jax.ShapeDtypeStruct((NW, P_PAD), jnp.float32),
            jax.ShapeDtypeStruct((NW, P_PAD), jnp.int32),
        ],
        mesh=_mesh,
        compiler_params=pltpu.CompilerParams(needs_layout_passes=False, use_tc_tiling_on_sc=False),
        scratch_types=[
            pltpu.VMEM((NODE_PER_TILE,), jnp.float32),
            pltpu.VMEM((NODE_PER_TILE,), jnp.int32),
            pltpu.VMEM((P_PAD,), jnp.float32),
            pltpu.VMEM((P_PAD,), jnp.int32),
            pltpu.VMEM((L,), jnp.float32),
            pltpu.VMEM((L,), jnp.int32),
            pltpu.VMEM((L,), jnp.int32),
        ],
    )(beta_pad, par_pad)


def _combine_body(outb_hbm, outi_hbm, ntab_hbm, ptab_hbm, misc_hbm,
                  bparts, iparts, cpbuf, bbuf, rows, accbuf, sem):
    tid = _wid()
    pbase = tid * P_PER_TILE
    handles = []
    for t in range(NW):
        handles.append(pltpu.async_copy(
            outb_hbm.at[t, pl.ds(pbase, P_PER_TILE)], bparts.at[t], sem))
        handles.append(pltpu.async_copy(
            outi_hbm.at[t, pl.ds(pbase, P_PER_TILE)], iparts.at[t], sem))
    for h in handles:
        h.wait()

    for k in range(PCHUNKS):
        sl = pl.ds(k * L, L)
        b = bparts[0, sl]
        i = iparts[0, sl]
        for t in range(1, NW):
            bt = bparts[t, sl]
            it = iparts[t, sl]
            better = _lex_better(bt, it, b, i)
            b = jnp.where(better, bt, b)
            i = jnp.where(better, it, i)
        cpbuf[k // (PCHUNKS // 2), pl.ds((k % (PCHUNKS // 2)) * L, L)] = (
            jnp.clip(i, 0, N - 1))
        bbuf[sl] = b

    half = P_PER_TILE // 2
    h1 = pltpu.async_copy(ntab_hbm.at[cpbuf.at[0]], rows.at[pl.ds(0, half)], sem)
    h2 = pltpu.async_copy(ntab_hbm.at[cpbuf.at[1]], rows.at[pl.ds(half, half)], sem)
    h1.wait()
    h2.wait()

    iota = lax.iota(jnp.int32, L)
    col3 = jnp.full((L,), 3, jnp.int32)
    acc_h = jnp.zeros((L,), jnp.float32)
    acc_o = jnp.zeros((L,), jnp.float32)
    for k in range(PCHUNKS):
        r16 = k * L + iota
        b = bbuf[pl.ds(k * L, L)]
        has = b >= 0.0
        q16 = plsc.load_gather(rows, [r16, col3])
        plsc.store_scatter(rows, [r16, col3], jnp.where(has, q16, 0.0))
        acc_h = acc_h + jnp.where(has, 1.0, 0.0)
        acc_o = acc_o + jnp.where(has, 1.0 - b, 0.0)
    accbuf[0, ...] = acc_h
    accbuf[1, ...] = acc_o
    pltpu.sync_copy(rows, ptab_hbm.at[pl.ds(pbase, P_PER_TILE)])
    pltpu.sync_copy(accbuf, misc_hbm.at[tid])


def _combine(outb, outi, node_tab):
    return pl.kernel(
        _combine_body,
        out_type=[
            jax.ShapeDtypeStruct((P_PAD, 8), jnp.float32),
            jax.ShapeDtypeStruct((NW, 2, L), jnp.float32),
        ],
        mesh=_mesh,
        compiler_params=pltpu.CompilerParams(needs_layout_passes=False, use_tc_tiling_on_sc=False),
        scratch_types=[
            pltpu.VMEM((NW, P_PER_TILE), jnp.float32),
            pltpu.VMEM((NW, P_PER_TILE), jnp.int32),
            pltpu.VMEM((2, P_PER_TILE // 2), jnp.int32),
            pltpu.VMEM((P_PER_TILE,), jnp.float32),
            pltpu.VMEM((P_PER_TILE, 8), jnp.float32),
            pltpu.VMEM((2, L), jnp.float32),
            pltpu.SemaphoreType.DMA,
        ],
    )(outb, outi, node_tab)


def _rsqrt(a):
    bits = plsc.bitcast(a, jnp.int32)
    y = 0x5F3759DF - lax.shift_right_arithmetic(bits, 1)
    r = plsc.bitcast(y, jnp.float32)
    for _ in range(3):
        r = r * (1.5 - 0.5 * a * r * r)
    return r


def _edge_body(ntab_hbm, ptab_hbm, esrc_hbm, edst_hbm, parts_hbm,
               ptab_v, srcb, dstb, rows, accst, sem):
    wid = _wid()
    ebase = wid * EDGES_PER_TILE
    pltpu.sync_copy(ptab_hbm, ptab_v)

    iota = lax.iota(jnp.int32, L)
    cols = [jnp.full((L,), j, jnp.int32) for j in range(5)]

    def body(ci, acc):
        b0 = ebase + ci * ECHUNK
        pltpu.sync_copy(esrc_hbm.at[pl.ds(b0, ECHUNK)], srcb)
        pltpu.sync_copy(edst_hbm.at[pl.ds(b0, ECHUNK)], dstb)
        pltpu.async_copy(ntab_hbm.at[srcb], rows, sem).wait()
        for s5 in range(ESUB):
            r16 = s5 * L + iota
            dst16 = dstb[pl.ds(s5 * L, L)]
            sx0 = plsc.load_gather(rows, [r16, cols[0]])
            sx1 = plsc.load_gather(rows, [r16, cols[1]])
            sx2 = plsc.load_gather(rows, [r16, cols[2]])
            sq = plsc.load_gather(rows, [r16, cols[3]])
            sp = plsc.load_gather(rows, [r16, cols[4]])
            px0 = plsc.load_gather(ptab_v, [dst16, cols[0]])
            px1 = plsc.load_gather(ptab_v, [dst16, cols[1]])
            px2 = plsc.load_gather(ptab_v, [dst16, cols[2]])
            pq = plsc.load_gather(ptab_v, [dst16, cols[3]])
            label = sp == dst16.astype(jnp.float32)
            d0 = px0 - sx0
            d1 = px1 - sx1
            d2 = px2 - sx2
            dist2 = d0 * d0 + d1 * d1 + d2 * d2 + 1e-12
            dist = dist2 * _rsqrt(dist2)
            rep = jnp.maximum(2.0 - dist, 0.0)
            v = jnp.where(label, 3.0 * dist2, rep) * pq
            acc = acc + sq * v
        return acc

    acc = lax.fori_loop(0, NECHUNK, body, jnp.zeros((L,), jnp.float32))
    accst[...] = acc
    pltpu.sync_copy(accst, parts_hbm.at[wid])


def _edge_pass(node_tab, ptab, edge_src, edge_dst):
    return pl.kernel(
        _edge_body,
        out_type=jax.ShapeDtypeStruct((NW, L), jnp.float32),
        mesh=_mesh,
        compiler_params=pltpu.CompilerParams(needs_layout_passes=False, use_tc_tiling_on_sc=False),
        scratch_types=[
            pltpu.VMEM((P_PAD, 8), jnp.float32),
            pltpu.VMEM((ECHUNK,), jnp.int32),
            pltpu.VMEM((ECHUNK,), jnp.int32),
            pltpu.VMEM((ECHUNK, 8), jnp.float32),
            pltpu.VMEM((L,), jnp.float32),
            pltpu.SemaphoreType.DMA,
        ],
    )(node_tab, ptab, edge_src, edge_dst)


def _final_body(bsum_ref, misc_h_ref, misc_o_ref, parts_ref, out_ref):
    ltot = jnp.sum(parts_ref[...])
    has = jnp.sum(misc_h_ref[...])
    omb = jnp.sum(misc_o_ref[...])
    bsum = jnp.sum(bsum_ref[...])
    n_obj = jnp.maximum(has, 1.0)
    out = ltot / N + omb / n_obj + S_B * bsum / N
    out_ref[...] = out.reshape(1, 1)


def _final(bsum, misc_h, misc_o, parts):
    return pl.pallas_call(
        _final_body,
        out_shape=jax.ShapeDtypeStruct((1, 1), jnp.float32),
    )(bsum, misc_h, misc_o, parts)


def kernel(x, beta, node_parent, edge_src, edge_dst):
    pad_n = N_PAD - N
    beta_pad = jnp.pad(beta, (0, pad_n))
    par_pad = jnp.pad(node_parent.astype(jnp.int32), (0, pad_n))
    x_pad = jnp.pad(x, ((0, pad_n), (0, 0)))

    q2d, bsum = _prep(beta_pad.reshape(ROWS0, 128))
    q = q2d.reshape(N_PAD)

    node_tab = jnp.concatenate(
        [x_pad, q[:, None], par_pad.astype(jnp.float32)[:, None],
         jnp.zeros((N_PAD, 3), jnp.float32)], axis=1)

    outb, outi = _node_pass(beta_pad, par_pad)
    ptab, misc = _combine(outb, outi, node_tab)
    parts = _edge_pass(node_tab, ptab,
                       edge_src.astype(jnp.int32), edge_dst.astype(jnp.int32))
    out = _final(bsum, misc[:, 0, :], misc[:, 1, :], parts)
    return out[0, 0]


# speedup vs baseline: 151.8573x; 5.1282x over previous
import functools

import jax
import jax.numpy as jnp
from jax import lax
from jax.experimental import pallas as pl
from jax.experimental.pallas import tpu as pltpu
from jax.experimental.pallas import tpu_sc as plsc

N = 100000
P = 6250
E = 1600000
QMIN = 0.1
S_B = 1.0

NC = 2
NS = 16
NW = NC * NS
L = 16

NODE_PER_TILE = 3136
N_PAD = NODE_PER_TILE * NW
ROWS0 = N_PAD // 128

P_PER_TILE = 224
P_PAD = P_PER_TILE * NW
PCHUNKS = P_PER_TILE // L

EDGES_PER_TILE = E // NW
ECHUNK = 80
ESUB = ECHUNK // L
EBLOCK = 10000
NBLOCK = EDGES_PER_TILE // EBLOCK
RING = 5
NGROUP = EBLOCK // (ECHUNK * RING)

BIG_I = 1 << 30

_mesh = plsc.VectorSubcoreMesh(core_axis_name="c", subcore_axis_name="s")


def _wid():
    return lax.axis_index("c") * NS + lax.axis_index("s")


def _lex_better(b_new, i_new, b_cur, i_cur):
    return (b_new > b_cur) | ((b_new == b_cur) & (i_new < i_cur))


def _prep_body(beta_ref, q_ref, bsum_ref):
    b = beta_ref[...]
    rows = lax.broadcasted_iota(jnp.int32, b.shape, 0)
    cols = lax.broadcasted_iota(jnp.int32, b.shape, 1)
    valid = (rows * 128 + cols) < N
    bc = jnp.clip(b, 0.0, 1.0 - 1e-4)
    at = 0.5 * jnp.log((1.0 + bc) / (1.0 - bc))
    q = at * at + QMIN
    q_ref[...] = jnp.where(valid, q, 0.0)
    bsum_ref[...] = jnp.sum(jnp.where(valid, bc, 0.0)).reshape(1, 1)


def _prep(beta2d):
    return pl.pallas_call(
        _prep_body,
        out_shape=[
            jax.ShapeDtypeStruct((ROWS0, 128), jnp.float32),
            jax.ShapeDtypeStruct((1, 1), jnp.float32),
        ],
    )(beta2d)


def _node_pass_body(beta_hbm, par_hbm, outb_hbm, outi_hbm,
                    beta_s, par_s, tab_b, tab_i, pbuf_f, pbuf_i, ibuf):
    wid = _wid()
    base = wid * NODE_PER_TILE
    pltpu.sync_copy(beta_hbm.at[pl.ds(base, NODE_PER_TILE)], beta_s)
    pltpu.sync_copy(par_hbm.at[pl.ds(base, NODE_PER_TILE)], par_s)

    iota = lax.iota(jnp.int32, L)
    neg1 = jnp.full((L,), -1.0, jnp.float32)
    big = jnp.full((L,), BIG_I, jnp.int32)

    def init_body(k, carry):
        tab_b[pl.ds(k * L, L)] = neg1
        tab_i[pl.ds(k * L, L)] = big
        return carry

    lax.fori_loop(0, P_PAD // L, init_body, 0)

    def body(ci, carry):
        off = ci * L
        b_raw = beta_s[pl.ds(off, L)]
        p_raw = par_s[pl.ds(off, L)]
        gidx = base + off + iota
        valid = gidx < N
        b16 = jnp.where(valid, jnp.clip(b_raw, 0.0, 1.0 - 1e-4), -1.0)
        p16 = jnp.where(valid, p_raw, 0)
        pbuf_f[...] = b16
        pbuf_i[...] = p16
        ibuf[...] = gidx
        lose = jnp.zeros((L,), jnp.bool_)
        for j in range(L):
            sj = jnp.full((L,), j, jnp.int32)
            pj = plsc.load_gather(pbuf_i, [sj])
            bj = plsc.load_gather(pbuf_f, [sj])
            ij = plsc.load_gather(ibuf, [sj])
            lose = lose | ((pj == p16) & _lex_better(bj, ij, b16, gidx))
        tb = plsc.load_gather(tab_b, [p16])
        ti = plsc.load_gather(tab_i, [p16])
        upd = (~lose) & _lex_better(b16, gidx, tb, ti)
        plsc.store_scatter(tab_b, [p16], b16, mask=upd)
        plsc.store_scatter(tab_i, [p16], gidx, mask=upd)
        return carry

    lax.fori_loop(0, NODE_PER_TILE // L, body, 0)
    pltpu.sync_copy(tab_b, outb_hbm.at[wid])
    pltpu.sync_copy(tab_i, outi_hbm.at[wid])


def _node_pass(beta_pad, par_pad):
    return pl.kernel(
        _node_pass_body,
        out_type=[
            jax.ShapeDtypeStruct((NW, P_PAD), jnp.float32),
            jax.ShapeDtypeStruct((NW, P_PAD), jnp.int32),
        ],
        mesh=_mesh,
        compiler_params=pltpu.CompilerParams(needs_layout_passes=False, use_tc_tiling_on_sc=False),
        scratch_types=[
            pltpu.VMEM((NODE_PER_TILE,), jnp.float32),
            pltpu.VMEM((NODE_PER_TILE,), jnp.int32),
            pltpu.VMEM((P_PAD,), jnp.float32),
            pltpu.VMEM((P_PAD,), jnp.int32),
            pltpu.VMEM((L,), jnp.float32),
            pltpu.VMEM((L,), jnp.int32),
            pltpu.VMEM((L,), jnp.int32),
        ],
    )(beta_pad, par_pad)


def _combine_body(outb_hbm, outi_hbm, ntab_hbm, ptab_hbm, misc_hbm,
                  bparts, iparts, cpbuf, bbuf, rows, accbuf, sem):
    tid = _wid()
    pbase = tid * P_PER_TILE
    handles = []
    for t in range(NW):
        handles.append(pltpu.async_copy(
            outb_hbm.at[t, pl.ds(pbase, P_PER_TILE)], bparts.at[t], sem))
        handles.append(pltpu.async_copy(
            outi_hbm.at[t, pl.ds(pbase, P_PER_TILE)], iparts.at[t], sem))
    for h in handles:
        h.wait()

    for k in range(PCHUNKS):
        sl = pl.ds(k * L, L)
        b = bparts[0, sl]
        i = iparts[0, sl]
        for t in range(1, NW):
            bt = bparts[t, sl]
            it = iparts[t, sl]
            better = _lex_better(bt, it, b, i)
            b = jnp.where(better, bt, b)
            i = jnp.where(better, it, i)
        cpbuf[k // (PCHUNKS // 2), pl.ds((k % (PCHUNKS // 2)) * L, L)] = (
            jnp.clip(i, 0, N - 1))
        bbuf[sl] = b

    half = P_PER_TILE // 2
    h1 = pltpu.async_copy(ntab_hbm.at[cpbuf.at[0]], rows.at[pl.ds(0, half)], sem)
    h2 = pltpu.async_copy(ntab_hbm.at[cpbuf.at[1]], rows.at[pl.ds(half, half)], sem)
    h1.wait()
    h2.wait()

    iota = lax.iota(jnp.int32, L)
    col3 = jnp.full((L,), 3, jnp.int32)
    acc_h = jnp.zeros((L,), jnp.float32)
    acc_o = jnp.zeros((L,), jnp.float32)
    for k in range(PCHUNKS):
        r16 = k * L + iota
        b = bbuf[pl.ds(k * L, L)]
        has = b >= 0.0
        q16 = plsc.load_gather(rows, [r16, col3])
        plsc.store_scatter(rows, [r16, col3], jnp.where(has, q16, 0.0))
        acc_h = acc_h + jnp.where(has, 1.0, 0.0)
        acc_o = acc_o + jnp.where(has, 1.0 - b, 0.0)
    accbuf[0, ...] = acc_h
    accbuf[1, ...] = acc_o
    pltpu.sync_copy(rows, ptab_hbm.at[pl.ds(pbase, P_PER_TILE)])
    pltpu.sync_copy(accbuf, misc_hbm.at[tid])


def _combine(outb, outi, node_tab):
    return pl.kernel(
        _combine_body,
        out_type=[
            jax.ShapeDtypeStruct((P_PAD, 8), jnp.float32),
            jax.ShapeDtypeStruct((NW, 2, L), jnp.float32),
        ],
        mesh=_mesh,
        compiler_params=pltpu.CompilerParams(needs_layout_passes=False, use_tc_tiling_on_sc=False),
        scratch_types=[
            pltpu.VMEM((NW, P_PER_TILE), jnp.float32),
            pltpu.VMEM((NW, P_PER_TILE), jnp.int32),
            pltpu.VMEM((2, P_PER_TILE // 2), jnp.int32),
            pltpu.VMEM((P_PER_TILE,), jnp.float32),
            pltpu.VMEM((P_PER_TILE, 8), jnp.float32),
            pltpu.VMEM((2, L), jnp.float32),
            pltpu.SemaphoreType.DMA,
        ],
    )(outb, outi, node_tab)


def _rsqrt(a):
    bits = plsc.bitcast(a, jnp.int32)
    y = 0x5F3759DF - lax.shift_right_arithmetic(bits, 1)
    r = plsc.bitcast(y, jnp.float32)
    for _ in range(3):
        r = r * (1.5 - 0.5 * a * r * r)
    return r


def _edge_body(ntab_hbm, ptab_hbm, esrc_hbm, edst_hbm, parts_hbm,
               ptab_v, srcb, dstb, rows, accst, sem_p, sem_e, sem_g):
    wid = _wid()
    ebase = wid * EDGES_PER_TILE

    hp = pltpu.async_copy(ptab_hbm, ptab_v, sem_p)

    def fire_block(b, par):
        b0 = ebase + b * EBLOCK
        h1 = pltpu.async_copy(esrc_hbm.at[pl.ds(b0, EBLOCK)], srcb.at[par],
                              sem_e)
        h2 = pltpu.async_copy(edst_hbm.at[pl.ds(b0, EBLOCK)], dstb.at[par],
                              sem_e)
        return [h1, h2]

    pend = fire_block(0, 0)
    hp.wait()

    iota = lax.iota(jnp.int32, L)
    cols = [jnp.full((L,), j, jnp.int32) for j in range(5)]
    acc = jnp.zeros((L,), jnp.float32)

    def gather(par, c, r):
        return pltpu.async_copy(
            ntab_hbm.at[srcb.at[par, pl.ds(c * ECHUNK, ECHUNK)]],
            rows.at[r], sem_g.at[r])

    def wait_slot(r):
        pltpu.make_async_copy(ntab_hbm.at[pl.ds(0, ECHUNK)],
                              rows.at[r], sem_g.at[r]).wait()

    for b in range(NBLOCK):
        par = b % 2
        for h in pend:
            h.wait()
        pend = fire_block(b + 1, (b + 1) % 2) if b + 1 < NBLOCK else []
        for r in range(RING):
            gather(par, r, r)

        def body(k, acc):
            for r in range(RING):
                c = k * RING + r
                wait_slot(r)
                for s5 in range(ESUB):
                    r16 = s5 * L + iota
                    dst16 = dstb[par, pl.ds(c * ECHUNK + s5 * L, L)]
                    rref = rows.at[r]
                    sx0 = plsc.load_gather(rref, [r16, cols[0]])
                    sx1 = plsc.load_gather(rref, [r16, cols[1]])
                    sx2 = plsc.load_gather(rref, [r16, cols[2]])
                    sq = plsc.load_gather(rref, [r16, cols[3]])
                    sp = plsc.load_gather(rref, [r16, cols[4]])
                    px0 = plsc.load_gather(ptab_v, [dst16, cols[0]])
                    px1 = plsc.load_gather(ptab_v, [dst16, cols[1]])
                    px2 = plsc.load_gather(ptab_v, [dst16, cols[2]])
                    pq = plsc.load_gather(ptab_v, [dst16, cols[3]])
                    label = sp == dst16.astype(jnp.float32)
                    d0 = px0 - sx0
                    d1 = px1 - sx1
                    d2 = px2 - sx2
                    dist2 = d0 * d0 + d1 * d1 + d2 * d2 + 1e-12
                    dist = dist2 * _rsqrt(dist2)
                    rep = jnp.maximum(2.0 - dist, 0.0)
                    v = jnp.where(label, 3.0 * dist2, rep) * pq
                    acc = acc + sq * v

                @pl.when(k < NGROUP - 1)
                def _():
                    gather(par, c + RING, r)

            return acc

        acc = lax.fori_loop(0, NGROUP, body, acc)

    accst[...] = acc
    pltpu.sync_copy(accst, parts_hbm.at[wid])


def _edge_pass(node_tab, ptab, edge_src, edge_dst):
    return pl.kernel(
        _edge_body,
        out_type=jax.ShapeDtypeStruct((NW, L), jnp.float32),
        mesh=_mesh,
        compiler_params=pltpu.CompilerParams(needs_layout_passes=False, use_tc_tiling_on_sc=False),
        scratch_types=[
            pltpu.VMEM((P_PAD, 8), jnp.float32),
            pltpu.VMEM((2, EBLOCK), jnp.int32),
            pltpu.VMEM((2, EBLOCK), jnp.int32),
            pltpu.VMEM((RING, ECHUNK, 8), jnp.float32),
            pltpu.VMEM((L,), jnp.float32),
            pltpu.SemaphoreType.DMA,
            pltpu.SemaphoreType.DMA,
            pltpu.SemaphoreType.DMA((RING,)),
        ],
    )(node_tab, ptab, edge_src, edge_dst)


def _final_body(bsum_ref, misc_h_ref, misc_o_ref, parts_ref, out_ref):
    ltot = jnp.sum(parts_ref[...])
    has = jnp.sum(misc_h_ref[...])
    omb = jnp.sum(misc_o_ref[...])
    bsum = jnp.sum(bsum_ref[...])
    n_obj = jnp.maximum(has, 1.0)
    out = ltot / N + omb / n_obj + S_B * bsum / N
    out_ref[...] = out.reshape(1, 1)


def _final(bsum, misc_h, misc_o, parts):
    return pl.pallas_call(
        _final_body,
        out_shape=jax.ShapeDtypeStruct((1, 1), jnp.float32),
    )(bsum, misc_h, misc_o, parts)


def kernel(x, beta, node_parent, edge_src, edge_dst):
    pad_n = N_PAD - N
    beta_pad = jnp.pad(beta, (0, pad_n))
    par_pad = jnp.pad(node_parent.astype(jnp.int32), (0, pad_n))
    x_pad = jnp.pad(x, ((0, pad_n), (0, 0)))

    q2d, bsum = _prep(beta_pad.reshape(ROWS0, 128))
    q = q2d.reshape(N_PAD)

    node_tab = jnp.concatenate(
        [x_pad, q[:, None], par_pad.astype(jnp.float32)[:, None],
         jnp.zeros((N_PAD, 3), jnp.float32)], axis=1)

    outb, outi = _node_pass(beta_pad, par_pad)
    ptab, misc = _combine(outb, outi, node_tab)
    parts = _edge_pass(node_tab, ptab,
                       edge_src.astype(jnp.int32), edge_dst.astype(jnp.int32))
    out = _final(bsum, misc[:, 0, :], misc[:, 1, :], parts)
    return out[0, 0]
